# baseline (speedup 1.0000x reference)
import jax
import jax.numpy as jnp
from jax.experimental import pallas as pl
from jax.experimental.pallas import tpu as pltpu

_BM = 512
_BK = 512


def _sage_body(a_ref, x_ref, cnt_ref, xd_ref, wl_ref, wr_ref, bl_ref,
               o_ref, acc_ref):
    k = pl.program_id(1)

    @pl.when(k == 0)
    def _init():
        acc_ref[...] = jnp.zeros_like(acc_ref)

    acc_ref[...] += jnp.dot(a_ref[...], x_ref[...],
                            preferred_element_type=jnp.float32)

    @pl.when(k == pl.num_programs(1) - 1)
    def _epilogue():
        mean = (acc_ref[...] / jnp.maximum(cnt_ref[...], 1.0)
                ).astype(jnp.bfloat16)
        o_ref[...] = (jnp.dot(mean, wl_ref[...],
                              preferred_element_type=jnp.float32)
                      + jnp.dot(xd_ref[...], wr_ref[...],
                                preferred_element_type=jnp.float32)
                      + bl_ref[...]).astype(o_ref.dtype)


def _sage(a, x_src, cnt, x_dst, wl, wr, bias, out_dtype):
    np_, d = x_src.shape
    grid = (np_ // _BM, np_ // _BK)
    return pl.pallas_call(
        _sage_body,
        grid=grid,
        in_specs=[
            pl.BlockSpec((_BM, _BK), lambda m, k: (m, k)),
            pl.BlockSpec((_BK, d), lambda m, k: (k, 0)),
            pl.BlockSpec((_BM, 1), lambda m, k: (m, 0)),
            pl.BlockSpec((_BM, d), lambda m, k: (m, 0)),
            pl.BlockSpec((d, d), lambda m, k: (0, 0)),
            pl.BlockSpec((d, d), lambda m, k: (0, 0)),
            pl.BlockSpec((1, d), lambda m, k: (0, 0)),
        ],
        out_specs=pl.BlockSpec((_BM, d), lambda m, k: (m, 0)),
        out_shape=jax.ShapeDtypeStruct((np_, d), out_dtype),
        scratch_shapes=[pltpu.VMEM((_BM, d), jnp.float32)],
        compiler_params=pltpu.CompilerParams(
            dimension_semantics=("parallel", "arbitrary")),
    )(a, x_src, cnt, x_dst, wl, wr, bias)


def kernel(x_u, x_v, edge_index_uv, edge_index_vu, Wl, bl, Wr):
    n_u, d = x_u.shape
    n_v = x_v.shape[0]
    np_ = ((max(n_u, n_v) + _BM - 1) // _BM) * _BM

    xu = jnp.zeros((np_, d), jnp.bfloat16).at[:n_u].set(
        x_u.astype(jnp.bfloat16))
    xv = jnp.zeros((np_, d), jnp.bfloat16).at[:n_v].set(
        x_v.astype(jnp.bfloat16))

    a_uv = jnp.zeros((np_, np_), jnp.bfloat16).at[
        edge_index_uv[1], edge_index_uv[0]].add(jnp.bfloat16(1.0))
    a_vu = jnp.zeros((np_, np_), jnp.bfloat16).at[
        edge_index_vu[1], edge_index_vu[0]].add(jnp.bfloat16(1.0))
    cnt_v = jnp.zeros((np_, 1), jnp.float32).at[edge_index_uv[1], 0].add(1.0)
    cnt_u = jnp.zeros((np_, 1), jnp.float32).at[edge_index_vu[1], 0].add(1.0)

    wl16 = Wl.astype(jnp.bfloat16)
    wr16 = Wr.astype(jnp.bfloat16)
    num_layers = Wl.shape[0] // 2
    for i in range(num_layers):
        last = i == num_layers - 1
        odt = jnp.float32 if last else jnp.bfloat16
        new_v = _sage(a_uv, xu, cnt_v, xv, wl16[2 * i], wr16[2 * i],
                      bl[2 * i][None, :], odt)
        new_u = _sage(a_vu, xv, cnt_u, xu, wl16[2 * i + 1], wr16[2 * i + 1],
                      bl[2 * i + 1][None, :], odt)
        xu, xv = new_u, new_v
    return xu[:n_u], xv[:n_v]


# speedup vs baseline: 2.1084x; 2.1084x over previous
import jax
import jax.numpy as jnp
from jax import lax
from jax.experimental import pallas as pl
from jax.experimental.pallas import tpu as pltpu
from jax.experimental.pallas import tpu_sc as plsc

_NP = 5120
_E = 80000
_NT = 16
_ET = _E // _NT
_EIT = 313
_ETP = _EIT * 16
_ROWS = 256
_NCH = _NP // _ROWS
_CE = _ROWS * _NP
_JUNK = 2048
_CBUF = _CE + _JUNK
_TZ = _CE // _NT
_ZB = 16384
_NZ = _TZ // _ZB
_CNTB = 8192
_CPT = _CNTB // _NT
_NDMA = 40
_MARK = 6000

_BM = 512
_BK = 512


def _sc_build_one(ei_ref, a_ref, cnt_ref, tid, dst_v, src_v, flat_v,
                  dump_v, idx2d, cidx2d, zeros_v, ones_f, zf_v,
                  chunk_sh, cnt_sh, sem):
    pltpu.sync_copy(ei_ref.at[pl.ds(_E + tid * _ET, _ET)],
                    dst_v.at[pl.ds(0, _ET)])
    pltpu.sync_copy(ei_ref.at[pl.ds(tid * _ET, _ET)],
                    src_v.at[pl.ds(0, _ET)])

    lane = lax.iota(jnp.int32, 16)
    tail = 16 * (_EIT - 1)
    keep = lane < (_ET - tail)
    dst_v[pl.ds(tail, 16)] = jnp.where(keep, dst_v[pl.ds(tail, 16)], _MARK)
    src_v[pl.ds(tail, 16)] = jnp.where(keep, src_v[pl.ds(tail, 16)], 0)

    def _pre(i, c):
        d = dst_v[pl.ds(i * 16, 16)]
        s = src_v[pl.ds(i * 16, 16)]
        flat_v[pl.ds(i * 16, 16)] = d * _NP + s
        dump_v[pl.ds(i * 16, 16)] = _CE + (s & (_JUNK - 1))
        cidx2d[i // 8, pl.ds((i % 8) * 16, 16)] = d
        return c

    lax.fori_loop(0, _EIT, _pre, 0)

    for j in range(7):
        idx2d[_NDMA - 1, pl.ds(16 + j * 16, 16)] = jnp.full(
            (16,), _CE + j * 16, jnp.int32) + lane
        cidx2d[_NDMA - 1, pl.ds(16 + j * 16, 16)] = jnp.full(
            (16,), _MARK, jnp.int32)

    def _chunk(c, carry):
        cb = c * _CE
        plsc.subcore_barrier()
        for z in range(_NZ):
            pltpu.sync_copy(zeros_v,
                            chunk_sh.at[pl.ds(tid * _TZ + z * _ZB, _ZB)])
        plsc.subcore_barrier()

        def _idx(i, cc):
            rel = flat_v[pl.ds(i * 16, 16)] - cb
            ok = (rel >= 0) & (rel < _CE)
            idx2d[i // 8, pl.ds((i % 8) * 16, 16)] = jnp.where(
                ok, rel, dump_v[pl.ds(i * 16, 16)])
            return cc

        lax.fori_loop(0, _EIT, _idx, 0)
        descs = [pltpu.async_copy(ones_f, chunk_sh.at[idx2d.at[j]], sem,
                                  add=True) for j in range(_NDMA)]
        for dsc in descs:
            dsc.wait()
        plsc.subcore_barrier()
        pltpu.sync_copy(chunk_sh.at[pl.ds(tid * _TZ, _TZ)],
                        a_ref.at[pl.ds(cb + tid * _TZ, _TZ)])
        return carry

    lax.fori_loop(0, _NCH, _chunk, 0)

    plsc.subcore_barrier()
    pltpu.sync_copy(zf_v, cnt_sh.at[pl.ds(tid * _CPT, _CPT)])
    plsc.subcore_barrier()
    descs = [pltpu.async_copy(ones_f, cnt_sh.at[cidx2d.at[j]], sem,
                              add=True) for j in range(_NDMA)]
    for dsc in descs:
        dsc.wait()
    plsc.subcore_barrier()
    pltpu.sync_copy(cnt_sh.at[pl.ds(tid * _CPT, _CPT)],
                    cnt_ref.at[pl.ds(tid * _CPT, _CPT)])


def _sc_build_body(ei_uv_ref, ei_vu_ref, zeros_hbm_ref,
                   a_uv_ref, a_vu_ref, cnt_v_ref,
                   cnt_u_ref, dst_v, src_v, flat_v, dump_v, idx2d, cidx2d,
                   zeros_v, ones_f, zf_v, chunk_sh, cnt_sh, sem):
    cid = lax.axis_index("c")
    tid = lax.axis_index("s")

    pltpu.sync_copy(zeros_hbm_ref, zeros_v)
    for j in range(8):
        ones_f[pl.ds(j * 16, 16)] = jnp.ones((16,), jnp.float32)

    def _zf(i, c):
        zf_v[pl.ds(i * 16, 16)] = jnp.zeros((16,), jnp.float32)
        return c

    lax.fori_loop(0, _CPT // 16, _zf, 0)

    args = (tid, dst_v, src_v, flat_v, dump_v, idx2d, cidx2d, zeros_v,
            ones_f, zf_v, chunk_sh, cnt_sh, sem)

    @pl.when(cid == 0)
    def _():
        _sc_build_one(ei_uv_ref, a_uv_ref, cnt_v_ref, *args)

    @pl.when(cid == 1)
    def _():
        _sc_build_one(ei_vu_ref, a_vu_ref, cnt_u_ref, *args)


def _sc_build(ei_uv, ei_vu):
    f = pl.kernel(
        _sc_build_body,
        out_type=(
            jax.ShapeDtypeStruct((_NP * _NP,), jnp.float32),
            jax.ShapeDtypeStruct((_NP * _NP,), jnp.float32),
            jax.ShapeDtypeStruct((_CNTB,), jnp.float32),
            jax.ShapeDtypeStruct((_CNTB,), jnp.float32),
        ),
        mesh=plsc.VectorSubcoreMesh(core_axis_name="c",
                                    subcore_axis_name="s"),
        scratch_types=[
            pltpu.VMEM((_ETP,), jnp.int32),
            pltpu.VMEM((_ETP,), jnp.int32),
            pltpu.VMEM((_ETP,), jnp.int32),
            pltpu.VMEM((_ETP,), jnp.int32),
            pltpu.VMEM((_NDMA, 128), jnp.int32),
            pltpu.VMEM((_NDMA, 128), jnp.int32),
            pltpu.VMEM((_ZB,), jnp.float32),
            pltpu.VMEM((128,), jnp.float32),
            pltpu.VMEM((_CPT,), jnp.float32),
            pltpu.VMEM_SHARED((_CBUF,), jnp.float32),
            pltpu.VMEM_SHARED((_CNTB,), jnp.float32),
            pltpu.SemaphoreType.DMA,
        ],
    )
    zeros_hbm = jnp.zeros((_ZB,), jnp.float32)
    return f(ei_uv, ei_vu, zeros_hbm)


def _epilogue(acc_ref, cnt_ref, xd_ref, wl_ref, wr_ref, bl_ref, o_ref):
    mean = (acc_ref[...] / jnp.maximum(cnt_ref[...], 1.0)
            ).astype(jnp.bfloat16)
    o_ref[...] = (jnp.dot(mean, wl_ref[...],
                          preferred_element_type=jnp.float32)
                  + jnp.dot(xd_ref[...], wr_ref[...],
                            preferred_element_type=jnp.float32)
                  + bl_ref[...]).astype(o_ref.dtype)


def _sage_body(a_ref, x_ref, cnt_ref, xd_ref, wl_ref, wr_ref, bl_ref,
               o_ref, acc_ref):
    k = pl.program_id(1)

    @pl.when(k == 0)
    def _init():
        acc_ref[...] = jnp.zeros_like(acc_ref)

    acc_ref[...] += jnp.dot(a_ref[...], x_ref[...],
                            preferred_element_type=jnp.float32)

    @pl.when(k == pl.num_programs(1) - 1)
    def _fin():
        _epilogue(acc_ref, cnt_ref, xd_ref, wl_ref, wr_ref, bl_ref, o_ref)


def _sage_cast_body(a_ref, x_ref, cnt_ref, xd_ref, wl_ref, wr_ref, bl_ref,
                    o_ref, a16_ref, acc_ref):
    k = pl.program_id(1)

    @pl.when(k == 0)
    def _init():
        acc_ref[...] = jnp.zeros_like(acc_ref)

    a16 = a_ref[...].astype(jnp.bfloat16)
    a16_ref[...] = a16
    acc_ref[...] += jnp.dot(a16, x_ref[...],
                            preferred_element_type=jnp.float32)

    @pl.when(k == pl.num_programs(1) - 1)
    def _fin():
        _epilogue(acc_ref, cnt_ref, xd_ref, wl_ref, wr_ref, bl_ref, o_ref)


def _sage(a, x_src, cnt, x_dst, wl, wr, bias, out_dtype, cast_a=False):
    np_, d = x_src.shape
    grid = (np_ // _BM, np_ // _BK)
    out_spec = pl.BlockSpec((_BM, d), lambda m, k: (m, 0))
    out_shape = jax.ShapeDtypeStruct((np_, d), out_dtype)
    if cast_a:
        body = _sage_cast_body
        out_spec = [out_spec, pl.BlockSpec((_BM, _BK), lambda m, k: (m, k))]
        out_shape = [out_shape,
                     jax.ShapeDtypeStruct((np_, np_), jnp.bfloat16)]
    else:
        body = _sage_body
    return pl.pallas_call(
        body,
        grid=grid,
        in_specs=[
            pl.BlockSpec((_BM, _BK), lambda m, k: (m, k)),
            pl.BlockSpec((_BK, d), lambda m, k: (k, 0)),
            pl.BlockSpec((_BM, 1), lambda m, k: (m, 0)),
            pl.BlockSpec((_BM, d), lambda m, k: (m, 0)),
            pl.BlockSpec((d, d), lambda m, k: (0, 0)),
            pl.BlockSpec((d, d), lambda m, k: (0, 0)),
            pl.BlockSpec((1, d), lambda m, k: (0, 0)),
        ],
        out_specs=out_spec,
        out_shape=out_shape,
        scratch_shapes=[pltpu.VMEM((_BM, d), jnp.float32)],
        compiler_params=pltpu.CompilerParams(
            dimension_semantics=("parallel", "arbitrary")),
    )(a, x_src, cnt, x_dst, wl, wr, bias)


def kernel(x_u, x_v, edge_index_uv, edge_index_vu, Wl, bl, Wr):
    n_u, d = x_u.shape
    n_v = x_v.shape[0]
    np_ = _NP

    xu = jnp.zeros((np_, d), jnp.bfloat16).at[:n_u].set(
        x_u.astype(jnp.bfloat16))
    xv = jnp.zeros((np_, d), jnp.bfloat16).at[:n_v].set(
        x_v.astype(jnp.bfloat16))

    a_uv_f, a_vu_f, cnt_v_f, cnt_u_f = _sc_build(
        edge_index_uv.reshape(-1), edge_index_vu.reshape(-1))
    a_uv = a_uv_f.reshape(np_, np_)
    a_vu = a_vu_f.reshape(np_, np_)
    cnt_v = cnt_v_f[:np_, None]
    cnt_u = cnt_u_f[:np_, None]

    wl16 = Wl.astype(jnp.bfloat16)
    wr16 = Wr.astype(jnp.bfloat16)
    num_layers = Wl.shape[0] // 2
    for i in range(num_layers):
        last = i == num_layers - 1
        odt = jnp.float32 if last else jnp.bfloat16
        if i == 0:
            new_v, a_uv = _sage(a_uv, xu, cnt_v, xv, wl16[0], wr16[0],
                                bl[0][None, :], odt, cast_a=True)
            new_u, a_vu = _sage(a_vu, xv, cnt_u, xu, wl16[1], wr16[1],
                                bl[1][None, :], odt, cast_a=True)
        else:
            new_v = _sage(a_uv, xu, cnt_v, xv, wl16[2 * i], wr16[2 * i],
                          bl[2 * i][None, :], odt)
            new_u = _sage(a_vu, xv, cnt_u, xu, wl16[2 * i + 1],
                          wr16[2 * i + 1], bl[2 * i + 1][None, :], odt)
        xu, xv = new_u, new_v
    return xu[:n_u], xv[:n_v]


# speedup vs baseline: 3.0686x; 1.4554x over previous
import jax
import jax.numpy as jnp
from jax import lax
from jax.experimental import pallas as pl
from jax.experimental.pallas import tpu as pltpu
from jax.experimental.pallas import tpu_sc as plsc

_NP = 5120
_E = 80000
_NT = 16
_ET = _E // _NT
_EIT = 313
_ETP = _EIT * 16
_ROWS = 256
_NCH = _NP // _ROWS
_CE = _ROWS * _NP
_JUNK = 2048
_CBUF = _CE + _JUNK
_TZ = _CE // _NT
_ZB = 16384
_NZ = _TZ // _ZB
_CNTB = 8192
_CPT = _CNTB // _NT
_NDMA = 40
_MARK = 6000

_BM = 512
_BK = 512


def _sc_build_one(ei_ref, a_ref, cnt_ref, tid, dst_v, src_v, flat_v,
                  dump_v, idx2d, cidx2d, zeros_v, ones_f, zf_v,
                  chunk_sh, cnt_sh, sem):
    pltpu.sync_copy(ei_ref.at[pl.ds(_E + tid * _ET, _ET)],
                    dst_v.at[pl.ds(0, _ET)])
    pltpu.sync_copy(ei_ref.at[pl.ds(tid * _ET, _ET)],
                    src_v.at[pl.ds(0, _ET)])

    lane = lax.iota(jnp.int32, 16)
    tail = 16 * (_EIT - 1)
    keep = lane < (_ET - tail)
    dst_v[pl.ds(tail, 16)] = jnp.where(keep, dst_v[pl.ds(tail, 16)], _MARK)
    src_v[pl.ds(tail, 16)] = jnp.where(keep, src_v[pl.ds(tail, 16)], 0)

    def _pre(i, c):
        d = dst_v[pl.ds(i * 16, 16)]
        s = src_v[pl.ds(i * 16, 16)]
        flat_v[pl.ds(i * 16, 16)] = d * _NP + s
        dump_v[pl.ds(i * 16, 16)] = _CE + (s & (_JUNK - 1))
        cidx2d[i // 8, pl.ds((i % 8) * 16, 16)] = d
        return c

    lax.fori_loop(0, _EIT, _pre, 0)

    for j in range(7):
        idx2d[_NDMA - 1, pl.ds(16 + j * 16, 16)] = jnp.full(
            (16,), _CE + j * 16, jnp.int32) + lane
        cidx2d[_NDMA - 1, pl.ds(16 + j * 16, 16)] = jnp.full(
            (16,), _MARK, jnp.int32)

    def _chunk(c, carry):
        cb = c * _CE
        plsc.subcore_barrier()
        for z in range(_NZ):
            pltpu.sync_copy(zeros_v,
                            chunk_sh.at[pl.ds(tid * _TZ + z * _ZB, _ZB)])
        plsc.subcore_barrier()

        def _idx(i, cc):
            rel = flat_v[pl.ds(i * 16, 16)] - cb
            ok = (rel >= 0) & (rel < _CE)
            idx2d[i // 8, pl.ds((i % 8) * 16, 16)] = jnp.where(
                ok, rel, dump_v[pl.ds(i * 16, 16)])
            return cc

        lax.fori_loop(0, _EIT, _idx, 0)
        descs = [pltpu.async_copy(ones_f, chunk_sh.at[idx2d.at[j]], sem,
                                  add=True) for j in range(_NDMA)]
        for dsc in descs:
            dsc.wait()
        plsc.subcore_barrier()
        pltpu.sync_copy(chunk_sh.at[pl.ds(tid * _TZ, _TZ)],
                        a_ref.at[pl.ds(cb + tid * _TZ, _TZ)])
        return carry

    lax.fori_loop(0, _NCH, _chunk, 0)

    plsc.subcore_barrier()
    pltpu.sync_copy(zf_v, cnt_sh.at[pl.ds(tid * _CPT, _CPT)])
    plsc.subcore_barrier()
    descs = [pltpu.async_copy(ones_f, cnt_sh.at[cidx2d.at[j]], sem,
                              add=True) for j in range(_NDMA)]
    for dsc in descs:
        dsc.wait()
    plsc.subcore_barrier()
    pltpu.sync_copy(cnt_sh.at[pl.ds(tid * _CPT, _CPT)],
                    cnt_ref.at[pl.ds(tid * _CPT, _CPT)])


def _sc_build_body(ei_uv_ref, ei_vu_ref, zeros_hbm_ref,
                   a_uv_ref, a_vu_ref, cnt_v_ref,
                   cnt_u_ref, dst_v, src_v, flat_v, dump_v, idx2d, cidx2d,
                   zeros_v, ones_f, zf_v, chunk_sh, cnt_sh, sem):
    cid = lax.axis_index("c")
    tid = lax.axis_index("s")

    pltpu.sync_copy(zeros_hbm_ref, zeros_v)
    for j in range(8):
        ones_f[pl.ds(j * 16, 16)] = jnp.ones((16,), jnp.float32)

    def _zf(i, c):
        zf_v[pl.ds(i * 16, 16)] = jnp.zeros((16,), jnp.float32)
        return c

    lax.fori_loop(0, _CPT // 16, _zf, 0)

    args = (tid, dst_v, src_v, flat_v, dump_v, idx2d, cidx2d, zeros_v,
            ones_f, zf_v, chunk_sh, cnt_sh, sem)

    @pl.when(cid == 0)
    def _():
        _sc_build_one(ei_uv_ref, a_uv_ref, cnt_v_ref, *args)

    @pl.when(cid == 1)
    def _():
        _sc_build_one(ei_vu_ref, a_vu_ref, cnt_u_ref, *args)


def _sc_build(ei_uv, ei_vu):
    f = pl.kernel(
        _sc_build_body,
        out_type=(
            jax.ShapeDtypeStruct((_NP * _NP,), jnp.float32),
            jax.ShapeDtypeStruct((_NP * _NP,), jnp.float32),
            jax.ShapeDtypeStruct((_CNTB,), jnp.float32),
            jax.ShapeDtypeStruct((_CNTB,), jnp.float32),
        ),
        mesh=plsc.VectorSubcoreMesh(core_axis_name="c",
                                    subcore_axis_name="s"),
        scratch_types=[
            pltpu.VMEM((_ETP,), jnp.int32),
            pltpu.VMEM((_ETP,), jnp.int32),
            pltpu.VMEM((_ETP,), jnp.int32),
            pltpu.VMEM((_ETP,), jnp.int32),
            pltpu.VMEM((_NDMA, 128), jnp.int32),
            pltpu.VMEM((_NDMA, 128), jnp.int32),
            pltpu.VMEM((_ZB,), jnp.float32),
            pltpu.VMEM((128,), jnp.float32),
            pltpu.VMEM((_CPT,), jnp.float32),
            pltpu.VMEM_SHARED((_CBUF,), jnp.float32),
            pltpu.VMEM_SHARED((_CNTB,), jnp.float32),
            pltpu.SemaphoreType.DMA,
        ],
    )
    zeros_hbm = jnp.zeros((_ZB,), jnp.float32)
    return f(ei_uv, ei_vu, zeros_hbm)


def _finish(agg, cnt_ref, xd_ref, wl_ref, wr_ref, bl_ref, o_ref):
    mean = (agg / jnp.maximum(cnt_ref[...], 1.0)).astype(jnp.bfloat16)
    o_ref[...] = (jnp.dot(mean, wl_ref[...],
                          preferred_element_type=jnp.float32)
                  + jnp.dot(xd_ref[...], wr_ref[...],
                            preferred_element_type=jnp.float32)
                  + bl_ref[...]).astype(o_ref.dtype)


def _sage_body(a_ref, x_ref, cnt_ref, xd_ref, wl_ref, wr_ref, bl_ref,
               o_ref):
    agg = jnp.dot(a_ref[...], x_ref[...],
                  preferred_element_type=jnp.float32)
    _finish(agg, cnt_ref, xd_ref, wl_ref, wr_ref, bl_ref, o_ref)


def _sage_cast_body(a_ref, x_ref, cnt_ref, xd_ref, wl_ref, wr_ref, bl_ref,
                    o_ref, a16_ref):
    a16 = a_ref[...].astype(jnp.bfloat16)
    a16_ref[...] = a16
    agg = jnp.dot(a16, x_ref[...], preferred_element_type=jnp.float32)
    _finish(agg, cnt_ref, xd_ref, wl_ref, wr_ref, bl_ref, o_ref)


def _sage(a, x_src, cnt, x_dst, wl, wr, bias, out_dtype, cast_a=False):
    np_, d = x_src.shape
    grid = (np_ // _BM,)
    out_spec = pl.BlockSpec((_BM, d), lambda m: (m, 0))
    out_shape = jax.ShapeDtypeStruct((np_, d), out_dtype)
    if cast_a:
        body = _sage_cast_body
        out_spec = [out_spec, pl.BlockSpec((_BM, np_), lambda m: (m, 0))]
        out_shape = [out_shape,
                     jax.ShapeDtypeStruct((np_, np_), jnp.bfloat16)]
    else:
        body = _sage_body
    return pl.pallas_call(
        body,
        grid=grid,
        in_specs=[
            pl.BlockSpec((_BM, np_), lambda m: (m, 0)),
            pl.BlockSpec((np_, d), lambda m: (0, 0)),
            pl.BlockSpec((_BM, 1), lambda m: (m, 0)),
            pl.BlockSpec((_BM, d), lambda m: (m, 0)),
            pl.BlockSpec((d, d), lambda m: (0, 0)),
            pl.BlockSpec((d, d), lambda m: (0, 0)),
            pl.BlockSpec((1, d), lambda m: (0, 0)),
        ],
        out_specs=out_spec,
        out_shape=out_shape,
        compiler_params=pltpu.CompilerParams(
            dimension_semantics=("arbitrary",)),
    )(a, x_src, cnt, x_dst, wl, wr, bias)


def kernel(x_u, x_v, edge_index_uv, edge_index_vu, Wl, bl, Wr):
    n_u, d = x_u.shape
    n_v = x_v.shape[0]
    np_ = _NP

    xu = jnp.zeros((np_, d), jnp.bfloat16).at[:n_u].set(
        x_u.astype(jnp.bfloat16))
    xv = jnp.zeros((np_, d), jnp.bfloat16).at[:n_v].set(
        x_v.astype(jnp.bfloat16))

    a_uv_f, a_vu_f, cnt_v_f, cnt_u_f = _sc_build(
        edge_index_uv.reshape(-1), edge_index_vu.reshape(-1))
    a_uv = a_uv_f.reshape(np_, np_)
    a_vu = a_vu_f.reshape(np_, np_)
    cnt_v = cnt_v_f[:np_, None]
    cnt_u = cnt_u_f[:np_, None]

    wl16 = Wl.astype(jnp.bfloat16)
    wr16 = Wr.astype(jnp.bfloat16)
    num_layers = Wl.shape[0] // 2
    for i in range(num_layers):
        last = i == num_layers - 1
        odt = jnp.float32 if last else jnp.bfloat16
        if i == 0:
            new_v, a_uv = _sage(a_uv, xu, cnt_v, xv, wl16[0], wr16[0],
                                bl[0][None, :], odt, cast_a=True)
            new_u, a_vu = _sage(a_vu, xv, cnt_u, xu, wl16[1], wr16[1],
                                bl[1][None, :], odt, cast_a=True)
        else:
            new_v = _sage(a_uv, xu, cnt_v, xv, wl16[2 * i], wr16[2 * i],
                          bl[2 * i][None, :], odt)
            new_u = _sage(a_vu, xv, cnt_u, xu, wl16[2 * i + 1],
                          wr16[2 * i + 1], bl[2 * i + 1][None, :], odt)
        xu, xv = new_u, new_v
    return xu[:n_u], xv[:n_v]


# speedup vs baseline: 3.7090x; 1.2087x over previous
import jax
import jax.numpy as jnp
from jax import lax
from jax.experimental import pallas as pl
from jax.experimental.pallas import tpu as pltpu
from jax.experimental.pallas import tpu_sc as plsc

_NP = 5120
_NPC = _NP // 2
_E = 80000
_NT = 16
_ET = _E // _NT
_EIT = 313
_ETP = _EIT * 16
_ROWS = 320
_NCH = _NP // _ROWS
_CE = _ROWS * _NPC
_JUNK = 2048
_CBUF = _CE + _JUNK
_TZ = _CE // _NT
_ZB = 12800
_NZ = _TZ // _ZB
_CNTB = 8192
_CPT = _CNTB // _NT
_NDMA = 40
_MARK = 10240

_BM = 512
_BK = 512


def _sc_build_one(ei_ref, a_ref, cnt_ref, tid, dst_v, src_v, flat_v,
                  dump_v, idx2d, vals2d, cidx2d, zeros_v, ones_f, zf_v,
                  chunk_sh, cnt_sh, sem):
    pltpu.sync_copy(ei_ref.at[pl.ds(_E + tid * _ET, _ET)],
                    dst_v.at[pl.ds(0, _ET)])
    pltpu.sync_copy(ei_ref.at[pl.ds(tid * _ET, _ET)],
                    src_v.at[pl.ds(0, _ET)])

    lane = lax.iota(jnp.int32, 16)
    tail = 16 * (_EIT - 1)
    keep = lane < (_ET - tail)
    dst_v[pl.ds(tail, 16)] = jnp.where(keep, dst_v[pl.ds(tail, 16)], _MARK)
    src_v[pl.ds(tail, 16)] = jnp.where(keep, src_v[pl.ds(tail, 16)], 0)

    def _pre(i, c):
        d = dst_v[pl.ds(i * 16, 16)]
        s = src_v[pl.ds(i * 16, 16)]
        p = (d >> 1) + (d & 1) * _NPC
        flat_v[pl.ds(i * 16, 16)] = p * _NPC + (s >> 1)
        dump_v[pl.ds(i * 16, 16)] = _CE + (s & (_JUNK - 1))
        vals2d[i // 8, pl.ds((i % 8) * 16, 16)] = 1 + (s & 1) * 65535
        cidx2d[i // 8, pl.ds((i % 8) * 16, 16)] = p
        return c

    lax.fori_loop(0, _EIT, _pre, 0)

    for j in range(7):
        idx2d[_NDMA - 1, pl.ds(16 + j * 16, 16)] = jnp.full(
            (16,), _CE + j * 16, jnp.int32) + lane
        vals2d[_NDMA - 1, pl.ds(16 + j * 16, 16)] = jnp.zeros(
            (16,), jnp.int32)
        cidx2d[_NDMA - 1, pl.ds(16 + j * 16, 16)] = jnp.full(
            (16,), _NP, jnp.int32)

    def _chunk(c, carry):
        cb = c * _CE
        plsc.subcore_barrier()
        for z in range(_NZ):
            pltpu.sync_copy(zeros_v,
                            chunk_sh.at[pl.ds(tid * _TZ + z * _ZB, _ZB)])
        plsc.subcore_barrier()

        def _idx(i, cc):
            rel = flat_v[pl.ds(i * 16, 16)] - cb
            ok = (rel >= 0) & (rel < _CE)
            idx2d[i // 8, pl.ds((i % 8) * 16, 16)] = jnp.where(
                ok, rel, dump_v[pl.ds(i * 16, 16)])
            return cc

        lax.fori_loop(0, _EIT, _idx, 0)
        descs = [pltpu.async_copy(vals2d.at[j], chunk_sh.at[idx2d.at[j]],
                                  sem, add=True) for j in range(_NDMA)]
        for dsc in descs:
            dsc.wait()
        plsc.subcore_barrier()
        pltpu.sync_copy(chunk_sh.at[pl.ds(tid * _TZ, _TZ)],
                        a_ref.at[pl.ds(cb + tid * _TZ, _TZ)])
        return carry

    lax.fori_loop(0, _NCH, _chunk, 0)

    plsc.subcore_barrier()
    pltpu.sync_copy(zf_v, cnt_sh.at[pl.ds(tid * _CPT, _CPT)])
    plsc.subcore_barrier()
    descs = [pltpu.async_copy(ones_f, cnt_sh.at[cidx2d.at[j]], sem,
                              add=True) for j in range(_NDMA)]
    for dsc in descs:
        dsc.wait()
    plsc.subcore_barrier()
    pltpu.sync_copy(cnt_sh.at[pl.ds(tid * _CPT, _CPT)],
                    cnt_ref.at[pl.ds(tid * _CPT, _CPT)])


def _sc_build_body(ei_uv_ref, ei_vu_ref, zeros_hbm_ref,
                   a_uv_ref, a_vu_ref, cnt_v_ref,
                   cnt_u_ref, dst_v, src_v, flat_v, dump_v, idx2d, vals2d,
                   cidx2d, zeros_v, ones_f, zf_v, chunk_sh, cnt_sh, sem):
    cid = lax.axis_index("c")
    tid = lax.axis_index("s")

    pltpu.sync_copy(zeros_hbm_ref, zeros_v)
    for j in range(8):
        ones_f[pl.ds(j * 16, 16)] = jnp.ones((16,), jnp.float32)

    def _zf(i, c):
        zf_v[pl.ds(i * 16, 16)] = jnp.zeros((16,), jnp.float32)
        return c

    lax.fori_loop(0, _CPT // 16, _zf, 0)

    args = (tid, dst_v, src_v, flat_v, dump_v, idx2d, vals2d, cidx2d,
            zeros_v, ones_f, zf_v, chunk_sh, cnt_sh, sem)

    @pl.when(cid == 0)
    def _():
        _sc_build_one(ei_uv_ref, a_uv_ref, cnt_v_ref, *args)

    @pl.when(cid == 1)
    def _():
        _sc_build_one(ei_vu_ref, a_vu_ref, cnt_u_ref, *args)


def _sc_build(ei_uv, ei_vu):
    f = pl.kernel(
        _sc_build_body,
        out_type=(
            jax.ShapeDtypeStruct((_NP * _NPC,), jnp.int32),
            jax.ShapeDtypeStruct((_NP * _NPC,), jnp.int32),
            jax.ShapeDtypeStruct((_CNTB,), jnp.float32),
            jax.ShapeDtypeStruct((_CNTB,), jnp.float32),
        ),
        mesh=plsc.VectorSubcoreMesh(core_axis_name="c",
                                    subcore_axis_name="s"),
        scratch_types=[
            pltpu.VMEM((_ETP,), jnp.int32),
            pltpu.VMEM((_ETP,), jnp.int32),
            pltpu.VMEM((_ETP,), jnp.int32),
            pltpu.VMEM((_ETP,), jnp.int32),
            pltpu.VMEM((_NDMA, 128), jnp.int32),
            pltpu.VMEM((_NDMA, 128), jnp.int32),
            pltpu.VMEM((_NDMA, 128), jnp.int32),
            pltpu.VMEM((_ZB,), jnp.int32),
            pltpu.VMEM((128,), jnp.float32),
            pltpu.VMEM((_CPT,), jnp.float32),
            pltpu.VMEM_SHARED((_CBUF,), jnp.int32),
            pltpu.VMEM_SHARED((_CNTB,), jnp.float32),
            pltpu.SemaphoreType.DMA,
        ],
    )
    zeros_hbm = jnp.zeros((_ZB,), jnp.int32)
    return f(ei_uv, ei_vu, zeros_hbm)


def _finish(agg, cnt_ref, xd_ref, wl_ref, wr_ref, bl_ref, o_ref):
    mean = (agg / jnp.maximum(cnt_ref[...], 1.0)).astype(jnp.bfloat16)
    o_ref[...] = (jnp.dot(mean, wl_ref[...],
                          preferred_element_type=jnp.float32)
                  + jnp.dot(xd_ref[...], wr_ref[...],
                            preferred_element_type=jnp.float32)
                  + bl_ref[...]).astype(o_ref.dtype)


def _sage_body(a_ref, x_ref, cnt_ref, xd_ref, wl_ref, wr_ref, bl_ref,
               o_ref):
    agg = jnp.dot(a_ref[...], x_ref[...],
                  preferred_element_type=jnp.float32)
    _finish(agg, cnt_ref, xd_ref, wl_ref, wr_ref, bl_ref, o_ref)


def _sage_cast_body(a_ref, x_ref, cnt_ref, xd_ref, wl_ref, wr_ref, bl_ref,
                    o_ref, a16_ref):
    a = a_ref[...]
    lo = (a & 0xFFFF).astype(jnp.float32).astype(jnp.bfloat16)
    hi = (a >> 16).astype(jnp.float32).astype(jnp.bfloat16)
    a16 = jnp.concatenate([lo, hi], axis=1)
    a16_ref[...] = a16
    agg = jnp.dot(a16, x_ref[...], preferred_element_type=jnp.float32)
    _finish(agg, cnt_ref, xd_ref, wl_ref, wr_ref, bl_ref, o_ref)


def _sage(a, x_src, cnt, x_dst, wl, wr, bias, out_dtype, cast_a=False):
    np_, d = x_src.shape
    grid = (np_ // _BM,)
    out_spec = pl.BlockSpec((_BM, d), lambda m: (m, 0))
    out_shape = jax.ShapeDtypeStruct((np_, d), out_dtype)
    a_cols = a.shape[1]
    if cast_a:
        body = _sage_cast_body
        out_spec = [out_spec, pl.BlockSpec((_BM, np_), lambda m: (m, 0))]
        out_shape = [out_shape,
                     jax.ShapeDtypeStruct((np_, np_), jnp.bfloat16)]
    else:
        body = _sage_body
    return pl.pallas_call(
        body,
        grid=grid,
        in_specs=[
            pl.BlockSpec((_BM, a_cols), lambda m: (m, 0)),
            pl.BlockSpec((np_, d), lambda m: (0, 0)),
            pl.BlockSpec((_BM, 1), lambda m: (m, 0)),
            pl.BlockSpec((_BM, d), lambda m: (m, 0)),
            pl.BlockSpec((d, d), lambda m: (0, 0)),
            pl.BlockSpec((d, d), lambda m: (0, 0)),
            pl.BlockSpec((1, d), lambda m: (0, 0)),
        ],
        out_specs=out_spec,
        out_shape=out_shape,
        compiler_params=pltpu.CompilerParams(
            dimension_semantics=("arbitrary",)),
    )(a, x_src, cnt, x_dst, wl, wr, bias)


def kernel(x_u, x_v, edge_index_uv, edge_index_vu, Wl, bl, Wr):
    n_u, d = x_u.shape
    n_v = x_v.shape[0]
    np_ = _NP

    xu = jnp.zeros((np_, d), jnp.bfloat16).at[:n_u].set(
        x_u.astype(jnp.bfloat16))
    xv = jnp.zeros((np_, d), jnp.bfloat16).at[:n_v].set(
        x_v.astype(jnp.bfloat16))
    xu = jnp.concatenate([xu[0::2], xu[1::2]], axis=0)
    xv = jnp.concatenate([xv[0::2], xv[1::2]], axis=0)

    a_uv_f, a_vu_f, cnt_v_f, cnt_u_f = _sc_build(
        edge_index_uv.reshape(-1), edge_index_vu.reshape(-1))
    a_uv = a_uv_f.reshape(np_, _NPC)
    a_vu = a_vu_f.reshape(np_, _NPC)
    cnt_v = cnt_v_f[:np_, None]
    cnt_u = cnt_u_f[:np_, None]

    wl16 = Wl.astype(jnp.bfloat16)
    wr16 = Wr.astype(jnp.bfloat16)
    num_layers = Wl.shape[0] // 2
    for i in range(num_layers):
        last = i == num_layers - 1
        odt = jnp.float32 if last else jnp.bfloat16
        if i == 0:
            new_v, a_uv = _sage(a_uv, xu, cnt_v, xv, wl16[0], wr16[0],
                                bl[0][None, :], odt, cast_a=True)
            new_u, a_vu = _sage(a_vu, xv, cnt_u, xu, wl16[1], wr16[1],
                                bl[1][None, :], odt, cast_a=True)
        else:
            new_v = _sage(a_uv, xu, cnt_v, xv, wl16[2 * i], wr16[2 * i],
                          bl[2 * i][None, :], odt)
            new_u = _sage(a_vu, xv, cnt_u, xu, wl16[2 * i + 1],
                          wr16[2 * i + 1], bl[2 * i + 1][None, :], odt)
        xu, xv = new_u, new_v

    xu = jnp.stack([xu[:_NPC], xu[_NPC:]], axis=1).reshape(np_, d)
    xv = jnp.stack([xv[:_NPC], xv[_NPC:]], axis=1).reshape(np_, d)
    return xu[:n_u], xv[:n_v]


# speedup vs baseline: 4.2890x; 1.1564x over previous
import jax
import jax.numpy as jnp
from jax import lax
from jax.experimental import pallas as pl
from jax.experimental.pallas import tpu as pltpu
from jax.experimental.pallas import tpu_sc as plsc

_NP = 5120
_NPC = _NP // 2
_E = 80000
_NT = 16
_ET = _E // _NT
_EIT = 313
_ETP = _EIT * 16
_ROWS = 320
_NCH = _NP // _ROWS
_CE = _ROWS * _NPC
_JUNK = 2048
_CBUF = _CE + _JUNK
_TZ = _CE // _NT
_ZB = 12800
_NZ = _TZ // _ZB
_CNTB = 8192
_CPT = _CNTB // _NT
_NDMA = 40
_MARK = 6000

_BM = 512
_BK = 512


def _sc_build_one(ei_ref, a_ref, cnt_ref, tid, dst_v, src_v, flat_v,
                  dump_v, idx2d, vals2d, cidx2d, zeros_v, ones_f, zf_v,
                  chunk_sh, cnt_sh, sem):
    pltpu.sync_copy(ei_ref.at[pl.ds(_E + tid * _ET, _ET)],
                    dst_v.at[pl.ds(0, _ET)])
    pltpu.sync_copy(ei_ref.at[pl.ds(tid * _ET, _ET)],
                    src_v.at[pl.ds(0, _ET)])

    lane = lax.iota(jnp.int32, 16)
    tail = 16 * (_EIT - 1)
    keep = lane < (_ET - tail)
    dst_v[pl.ds(tail, 16)] = jnp.where(keep, dst_v[pl.ds(tail, 16)], _MARK)
    src_v[pl.ds(tail, 16)] = jnp.where(keep, src_v[pl.ds(tail, 16)], 0)

    def _pre(i, c):
        d = dst_v[pl.ds(i * 16, 16)]
        s = src_v[pl.ds(i * 16, 16)]
        m = jnp.where(s >= _NPC, 1, 0)
        flat_v[pl.ds(i * 16, 16)] = d * _NPC + (s - m * _NPC)
        dump_v[pl.ds(i * 16, 16)] = _CE + (s & (_JUNK - 1))
        vals2d[i // 8, pl.ds((i % 8) * 16, 16)] = 1 + m * 65535
        cidx2d[i // 8, pl.ds((i % 8) * 16, 16)] = d
        return c

    lax.fori_loop(0, _EIT, _pre, 0)

    for j in range(7):
        idx2d[_NDMA - 1, pl.ds(16 + j * 16, 16)] = jnp.full(
            (16,), _CE + j * 16, jnp.int32) + lane
        vals2d[_NDMA - 1, pl.ds(16 + j * 16, 16)] = jnp.zeros(
            (16,), jnp.int32)
        cidx2d[_NDMA - 1, pl.ds(16 + j * 16, 16)] = jnp.full(
            (16,), _MARK, jnp.int32)

    def _chunk(c, carry):
        cb = c * _CE
        plsc.subcore_barrier()
        for z in range(_NZ):
            pltpu.sync_copy(zeros_v,
                            chunk_sh.at[pl.ds(tid * _TZ + z * _ZB, _ZB)])
        plsc.subcore_barrier()

        def _idx(i, cc):
            rel = flat_v[pl.ds(i * 16, 16)] - cb
            ok = (rel >= 0) & (rel < _CE)
            idx2d[i // 8, pl.ds((i % 8) * 16, 16)] = jnp.where(
                ok, rel, dump_v[pl.ds(i * 16, 16)])
            return cc

        lax.fori_loop(0, _EIT, _idx, 0)
        descs = [pltpu.async_copy(vals2d.at[j], chunk_sh.at[idx2d.at[j]],
                                  sem, add=True) for j in range(_NDMA)]
        for dsc in descs:
            dsc.wait()
        plsc.subcore_barrier()
        pltpu.sync_copy(chunk_sh.at[pl.ds(tid * _TZ, _TZ)],
                        a_ref.at[pl.ds(cb + tid * _TZ, _TZ)])
        return carry

    lax.fori_loop(0, _NCH, _chunk, 0)

    plsc.subcore_barrier()
    pltpu.sync_copy(zf_v, cnt_sh.at[pl.ds(tid * _CPT, _CPT)])
    plsc.subcore_barrier()
    descs = [pltpu.async_copy(ones_f, cnt_sh.at[cidx2d.at[j]], sem,
                              add=True) for j in range(_NDMA)]
    for dsc in descs:
        dsc.wait()
    plsc.subcore_barrier()
    pltpu.sync_copy(cnt_sh.at[pl.ds(tid * _CPT, _CPT)],
                    cnt_ref.at[pl.ds(tid * _CPT, _CPT)])


def _sc_build_body(ei_uv_ref, ei_vu_ref, zeros_hbm_ref,
                   a_uv_ref, a_vu_ref, cnt_v_ref,
                   cnt_u_ref, dst_v, src_v, flat_v, dump_v, idx2d, vals2d,
                   cidx2d, zeros_v, ones_f, zf_v, chunk_sh, cnt_sh, sem):
    cid = lax.axis_index("c")
    tid = lax.axis_index("s")

    pltpu.sync_copy(zeros_hbm_ref, zeros_v)
    for j in range(8):
        ones_f[pl.ds(j * 16, 16)] = jnp.ones((16,), jnp.float32)

    def _zf(i, c):
        zf_v[pl.ds(i * 16, 16)] = jnp.zeros((16,), jnp.float32)
        return c

    lax.fori_loop(0, _CPT // 16, _zf, 0)

    args = (tid, dst_v, src_v, flat_v, dump_v, idx2d, vals2d, cidx2d,
            zeros_v, ones_f, zf_v, chunk_sh, cnt_sh, sem)

    @pl.when(cid == 0)
    def _():
        _sc_build_one(ei_uv_ref, a_uv_ref, cnt_v_ref, *args)

    @pl.when(cid == 1)
    def _():
        _sc_build_one(ei_vu_ref, a_vu_ref, cnt_u_ref, *args)


def _sc_build(ei_uv, ei_vu):
    f = pl.kernel(
        _sc_build_body,
        out_type=(
            jax.ShapeDtypeStruct((_NP * _NPC,), jnp.int32),
            jax.ShapeDtypeStruct((_NP * _NPC,), jnp.int32),
            jax.ShapeDtypeStruct((_CNTB,), jnp.float32),
            jax.ShapeDtypeStruct((_CNTB,), jnp.float32),
        ),
        mesh=plsc.VectorSubcoreMesh(core_axis_name="c",
                                    subcore_axis_name="s"),
        scratch_types=[
            pltpu.VMEM((_ETP,), jnp.int32),
            pltpu.VMEM((_ETP,), jnp.int32),
            pltpu.VMEM((_ETP,), jnp.int32),
            pltpu.VMEM((_ETP,), jnp.int32),
            pltpu.VMEM((_NDMA, 128), jnp.int32),
            pltpu.VMEM((_NDMA, 128), jnp.int32),
            pltpu.VMEM((_NDMA, 128), jnp.int32),
            pltpu.VMEM((_ZB,), jnp.int32),
            pltpu.VMEM((128,), jnp.float32),
            pltpu.VMEM((_CPT,), jnp.float32),
            pltpu.VMEM_SHARED((_CBUF,), jnp.int32),
            pltpu.VMEM_SHARED((_CNTB,), jnp.float32),
            pltpu.SemaphoreType.DMA,
        ],
    )
    zeros_hbm = jnp.zeros((_ZB,), jnp.int32)
    return f(ei_uv, ei_vu, zeros_hbm)


def _finish(agg, cnt_ref, xd_ref, wl_ref, wr_ref, bl_ref, o_ref):
    mean = (agg / jnp.maximum(cnt_ref[...], 1.0)).astype(jnp.bfloat16)
    o_ref[...] = (jnp.dot(mean, wl_ref[...],
                          preferred_element_type=jnp.float32)
                  + jnp.dot(xd_ref[...], wr_ref[...],
                            preferred_element_type=jnp.float32)
                  + bl_ref[...]).astype(o_ref.dtype)


def _sage_body(a_ref, x_ref, cnt_ref, xd_ref, wl_ref, wr_ref, bl_ref,
               o_ref):
    agg = jnp.dot(a_ref[...], x_ref[...],
                  preferred_element_type=jnp.float32)
    _finish(agg, cnt_ref, xd_ref, wl_ref, wr_ref, bl_ref, o_ref)


def _sage_cast_body(a_ref, x_ref, cnt_ref, xd_ref, wl_ref, wr_ref, bl_ref,
                    o_ref, a16_ref):
    a = a_ref[...]
    lo = (a & 0xFFFF).astype(jnp.float32).astype(jnp.bfloat16)
    hi = (a >> 16).astype(jnp.float32).astype(jnp.bfloat16)
    a16 = jnp.concatenate([lo, hi], axis=1)
    a16_ref[...] = a16
    agg = jnp.dot(a16, x_ref[...], preferred_element_type=jnp.float32)
    _finish(agg, cnt_ref, xd_ref, wl_ref, wr_ref, bl_ref, o_ref)


def _sage(a, x_src, cnt, x_dst, wl, wr, bias, out_dtype, cast_a=False):
    np_, d = x_src.shape
    grid = (np_ // _BM,)
    out_spec = pl.BlockSpec((_BM, d), lambda m: (m, 0))
    out_shape = jax.ShapeDtypeStruct((np_, d), out_dtype)
    a_cols = a.shape[1]
    if cast_a:
        body = _sage_cast_body
        out_spec = [out_spec, pl.BlockSpec((_BM, np_), lambda m: (m, 0))]
        out_shape = [out_shape,
                     jax.ShapeDtypeStruct((np_, np_), jnp.bfloat16)]
    else:
        body = _sage_body
    return pl.pallas_call(
        body,
        grid=grid,
        in_specs=[
            pl.BlockSpec((_BM, a_cols), lambda m: (m, 0)),
            pl.BlockSpec((np_, d), lambda m: (0, 0)),
            pl.BlockSpec((_BM, 1), lambda m: (m, 0)),
            pl.BlockSpec((_BM, d), lambda m: (m, 0)),
            pl.BlockSpec((d, d), lambda m: (0, 0)),
            pl.BlockSpec((d, d), lambda m: (0, 0)),
            pl.BlockSpec((1, d), lambda m: (0, 0)),
        ],
        out_specs=out_spec,
        out_shape=out_shape,
        compiler_params=pltpu.CompilerParams(
            dimension_semantics=("arbitrary",)),
    )(a, x_src, cnt, x_dst, wl, wr, bias)


def kernel(x_u, x_v, edge_index_uv, edge_index_vu, Wl, bl, Wr):
    n_u, d = x_u.shape
    n_v = x_v.shape[0]
    np_ = _NP

    xu = jnp.zeros((np_, d), jnp.bfloat16).at[:n_u].set(
        x_u.astype(jnp.bfloat16))
    xv = jnp.zeros((np_, d), jnp.bfloat16).at[:n_v].set(
        x_v.astype(jnp.bfloat16))

    a_uv_f, a_vu_f, cnt_v_f, cnt_u_f = _sc_build(
        edge_index_uv.reshape(-1), edge_index_vu.reshape(-1))
    a_uv = a_uv_f.reshape(np_, _NPC)
    a_vu = a_vu_f.reshape(np_, _NPC)
    cnt_v = cnt_v_f[:np_, None]
    cnt_u = cnt_u_f[:np_, None]

    wl16 = Wl.astype(jnp.bfloat16)
    wr16 = Wr.astype(jnp.bfloat16)
    num_layers = Wl.shape[0] // 2
    for i in range(num_layers):
        last = i == num_layers - 1
        odt = jnp.float32 if last else jnp.bfloat16
        if i == 0:
            new_v, a_uv = _sage(a_uv, xu, cnt_v, xv, wl16[0], wr16[0],
                                bl[0][None, :], odt, cast_a=True)
            new_u, a_vu = _sage(a_vu, xv, cnt_u, xu, wl16[1], wr16[1],
                                bl[1][None, :], odt, cast_a=True)
        else:
            new_v = _sage(a_uv, xu, cnt_v, xv, wl16[2 * i], wr16[2 * i],
                          bl[2 * i][None, :], odt)
            new_u = _sage(a_vu, xv, cnt_u, xu, wl16[2 * i + 1],
                          wr16[2 * i + 1], bl[2 * i + 1][None, :], odt)
        xu, xv = new_u, new_v
    return xu[:n_u], xv[:n_v]


# speedup vs baseline: 4.3532x; 1.0150x over previous
import jax
import jax.numpy as jnp
from jax import lax
from jax.experimental import pallas as pl
from jax.experimental.pallas import tpu as pltpu
from jax.experimental.pallas import tpu_sc as plsc

_NP = 5120
_NPC = _NP // 2
_E = 80000
_NT = 16
_ET = _E // _NT
_EIT = 313
_ETP = _EIT * 16
_ROWS = 320
_NCH = _NP // _ROWS
_CE = _ROWS * _NPC
_JUNK = 2048
_CBUF = _CE + _JUNK
_TZ = _CE // _NT
_ZB = 12800
_NZ = _TZ // _ZB
_CNTB = 8192
_CPT = _CNTB // _NT
_NDMA = 40
_MARK = 6000

_BM = 512
_BK = 512


def _sc_build_one(ei_ref, a_ref, cnt_ref, tid, dst_v, src_v, flat_v,
                  dump_v, idx2d, vals2d, cidx2d, zeros_v, ones_f, zf_v,
                  chunk_sh, cnt_sh, sem):
    pltpu.sync_copy(ei_ref.at[pl.ds(_E + tid * _ET, _ET)],
                    dst_v.at[pl.ds(0, _ET)])
    pltpu.sync_copy(ei_ref.at[pl.ds(tid * _ET, _ET)],
                    src_v.at[pl.ds(0, _ET)])

    lane = lax.iota(jnp.int32, 16)
    tail = 16 * (_EIT - 1)
    keep = lane < (_ET - tail)
    dst_v[pl.ds(tail, 16)] = jnp.where(keep, dst_v[pl.ds(tail, 16)], _MARK)
    src_v[pl.ds(tail, 16)] = jnp.where(keep, src_v[pl.ds(tail, 16)], 0)

    def _pre(i, c):
        d = dst_v[pl.ds(i * 16, 16)]
        s = src_v[pl.ds(i * 16, 16)]
        m = jnp.where(s >= _NPC, 1, 0)
        flat_v[pl.ds(i * 16, 16)] = d * _NPC + (s - m * _NPC)
        dump_v[pl.ds(i * 16, 16)] = _CE + (s & (_JUNK - 1))
        vals2d[i // 8, pl.ds((i % 8) * 16, 16)] = 1 + m * 65535
        cidx2d[i // 8, pl.ds((i % 8) * 16, 16)] = d
        return c

    lax.fori_loop(0, _EIT, _pre, 0)

    for j in range(7):
        idx2d[_NDMA - 1, pl.ds(16 + j * 16, 16)] = jnp.full(
            (16,), _CE + j * 16, jnp.int32) + lane
        vals2d[_NDMA - 1, pl.ds(16 + j * 16, 16)] = jnp.zeros(
            (16,), jnp.int32)
        cidx2d[_NDMA - 1, pl.ds(16 + j * 16, 16)] = jnp.full(
            (16,), _MARK, jnp.int32)

    def _chunk(c, carry):
        cb = c * _CE
        plsc.subcore_barrier()
        for z in range(_NZ):
            pltpu.sync_copy(zeros_v,
                            chunk_sh.at[pl.ds(tid * _TZ + z * _ZB, _ZB)])
        plsc.subcore_barrier()

        def _idx(i, cc):
            rel = flat_v[pl.ds(i * 16, 16)] - cb
            ok = (rel >= 0) & (rel < _CE)
            idx2d[i // 8, pl.ds((i % 8) * 16, 16)] = jnp.where(
                ok, rel, dump_v[pl.ds(i * 16, 16)])
            return cc

        lax.fori_loop(0, _EIT, _idx, 0)
        descs = [pltpu.async_copy(vals2d.at[j], chunk_sh.at[idx2d.at[j]],
                                  sem, add=True) for j in range(_NDMA)]
        for dsc in descs:
            dsc.wait()
        plsc.subcore_barrier()
        pltpu.sync_copy(chunk_sh.at[pl.ds(tid * _TZ, _TZ)],
                        a_ref.at[pl.ds(cb + tid * _TZ, _TZ)])
        return carry

    lax.fori_loop(0, _NCH, _chunk, 0)

    plsc.subcore_barrier()
    pltpu.sync_copy(zf_v, cnt_sh.at[pl.ds(tid * _CPT, _CPT)])
    plsc.subcore_barrier()
    descs = [pltpu.async_copy(ones_f, cnt_sh.at[cidx2d.at[j]], sem,
                              add=True) for j in range(_NDMA)]
    for dsc in descs:
        dsc.wait()
    plsc.subcore_barrier()
    pltpu.sync_copy(cnt_sh.at[pl.ds(tid * _CPT, _CPT)],
                    cnt_ref.at[pl.ds(tid * _CPT, _CPT)])


def _sc_build_body(ei_uv_ref, ei_vu_ref, zeros_hbm_ref,
                   a_uv_ref, a_vu_ref, cnt_v_ref,
                   cnt_u_ref, dst_v, src_v, flat_v, dump_v, idx2d, vals2d,
                   cidx2d, zeros_v, ones_f, zf_v, chunk_sh, cnt_sh, sem):
    cid = lax.axis_index("c")
    tid = lax.axis_index("s")

    pltpu.sync_copy(zeros_hbm_ref, zeros_v)
    for j in range(8):
        ones_f[pl.ds(j * 16, 16)] = jnp.ones((16,), jnp.float32)

    def _zf(i, c):
        zf_v[pl.ds(i * 16, 16)] = jnp.zeros((16,), jnp.float32)
        return c

    lax.fori_loop(0, _CPT // 16, _zf, 0)

    args = (tid, dst_v, src_v, flat_v, dump_v, idx2d, vals2d, cidx2d,
            zeros_v, ones_f, zf_v, chunk_sh, cnt_sh, sem)

    @pl.when(cid == 0)
    def _():
        _sc_build_one(ei_uv_ref, a_uv_ref, cnt_v_ref, *args)

    @pl.when(cid == 1)
    def _():
        _sc_build_one(ei_vu_ref, a_vu_ref, cnt_u_ref, *args)


def _sc_build(ei_uv, ei_vu):
    f = pl.kernel(
        _sc_build_body,
        out_type=(
            jax.ShapeDtypeStruct((_NP * _NPC,), jnp.int32),
            jax.ShapeDtypeStruct((_NP * _NPC,), jnp.int32),
            jax.ShapeDtypeStruct((_CNTB,), jnp.float32),
            jax.ShapeDtypeStruct((_CNTB,), jnp.float32),
        ),
        mesh=plsc.VectorSubcoreMesh(core_axis_name="c",
                                    subcore_axis_name="s"),
        scratch_types=[
            pltpu.VMEM((_ETP,), jnp.int32),
            pltpu.VMEM((_ETP,), jnp.int32),
            pltpu.VMEM((_ETP,), jnp.int32),
            pltpu.VMEM((_ETP,), jnp.int32),
            pltpu.VMEM((_NDMA, 128), jnp.int32),
            pltpu.VMEM((_NDMA, 128), jnp.int32),
            pltpu.VMEM((_NDMA, 128), jnp.int32),
            pltpu.VMEM((_ZB,), jnp.int32),
            pltpu.VMEM((128,), jnp.float32),
            pltpu.VMEM((_CPT,), jnp.float32),
            pltpu.VMEM_SHARED((_CBUF,), jnp.int32),
            pltpu.VMEM_SHARED((_CNTB,), jnp.float32),
            pltpu.SemaphoreType.DMA,
        ],
    )
    zeros_hbm = jnp.zeros((_ZB,), jnp.int32)
    return f(ei_uv, ei_vu, zeros_hbm)


def _finish(agg, cnt_ref, xd_ref, wl_ref, wr_ref, bl_ref, o_ref):
    mean = (agg / jnp.maximum(cnt_ref[...], 1.0)).astype(jnp.bfloat16)
    o_ref[...] = (jnp.dot(mean, wl_ref[...],
                          preferred_element_type=jnp.float32)
                  + jnp.dot(xd_ref[...], wr_ref[...],
                            preferred_element_type=jnp.float32)
                  + bl_ref[...]).astype(o_ref.dtype)


def _sage_body(a_ref, x_ref, cnt_ref, xd_ref, wl_ref, wr_ref, bl_ref,
               o_ref):
    a = a_ref[...]
    lo = (a & 0xFFFF).astype(jnp.float32).astype(jnp.bfloat16)
    hi = (a >> 16).astype(jnp.float32).astype(jnp.bfloat16)
    a16 = jnp.concatenate([lo, hi], axis=1)
    agg = jnp.dot(a16, x_ref[...], preferred_element_type=jnp.float32)
    _finish(agg, cnt_ref, xd_ref, wl_ref, wr_ref, bl_ref, o_ref)


def _sage(a, x_src, cnt, x_dst, wl, wr, bias, out_dtype):
    np_, d = x_src.shape
    grid = (np_ // _BM,)
    return pl.pallas_call(
        _sage_body,
        grid=grid,
        in_specs=[
            pl.BlockSpec((_BM, _NPC), lambda m: (m, 0)),
            pl.BlockSpec((np_, d), lambda m: (0, 0)),
            pl.BlockSpec((_BM, 1), lambda m: (m, 0)),
            pl.BlockSpec((_BM, d), lambda m: (m, 0)),
            pl.BlockSpec((d, d), lambda m: (0, 0)),
            pl.BlockSpec((d, d), lambda m: (0, 0)),
            pl.BlockSpec((1, d), lambda m: (0, 0)),
        ],
        out_specs=pl.BlockSpec((_BM, d), lambda m: (m, 0)),
        out_shape=jax.ShapeDtypeStruct((np_, d), out_dtype),
        compiler_params=pltpu.CompilerParams(
            dimension_semantics=("arbitrary",)),
    )(a, x_src, cnt, x_dst, wl, wr, bias)


def kernel(x_u, x_v, edge_index_uv, edge_index_vu, Wl, bl, Wr):
    n_u, d = x_u.shape
    n_v = x_v.shape[0]
    np_ = _NP

    xu = jnp.zeros((np_, d), jnp.bfloat16).at[:n_u].set(
        x_u.astype(jnp.bfloat16))
    xv = jnp.zeros((np_, d), jnp.bfloat16).at[:n_v].set(
        x_v.astype(jnp.bfloat16))

    a_uv_f, a_vu_f, cnt_v_f, cnt_u_f = _sc_build(
        edge_index_uv.reshape(-1), edge_index_vu.reshape(-1))
    a_uv = a_uv_f.reshape(np_, _NPC)
    a_vu = a_vu_f.reshape(np_, _NPC)
    cnt_v = cnt_v_f[:np_, None]
    cnt_u = cnt_u_f[:np_, None]

    wl16 = Wl.astype(jnp.bfloat16)
    wr16 = Wr.astype(jnp.bfloat16)
    num_layers = Wl.shape[0] // 2
    for i in range(num_layers):
        last = i == num_layers - 1
        odt = jnp.float32 if last else jnp.bfloat16
        new_v = _sage(a_uv, xu, cnt_v, xv, wl16[2 * i], wr16[2 * i],
                      bl[2 * i][None, :], odt)
        new_u = _sage(a_vu, xv, cnt_u, xu, wl16[2 * i + 1],
                      wr16[2 * i + 1], bl[2 * i + 1][None, :], odt)
        xu, xv = new_u, new_v
    return xu[:n_u], xv[:n_v]


# speedup vs baseline: 4.3949x; 1.0096x over previous
import jax
import jax.numpy as jnp
from jax import lax
from jax.experimental import pallas as pl
from jax.experimental.pallas import tpu as pltpu
from jax.experimental.pallas import tpu_sc as plsc

_NP = 5120
_NPC = _NP // 2
_E = 80000
_NT = 16
_ET = _E // _NT
_EIT = 313
_ETP = _EIT * 16
_ROWS = 320
_NCH = _NP // _ROWS
_CE = _ROWS * _NPC
_JUNK = 2048
_CBUF = _CE + _JUNK
_TZ = _CE // _NT
_ZB = 12800
_NZ = _TZ // _ZB
_CNTB = 8192
_CPT = _CNTB // _NT
_NDMA = 40
_MARK = 6000

_BM = 512
_BK = 512


def _sc_build_one(ei_ref, a_ref, cnt_ref, tid, dst_v, src_v, flat_v,
                  dump_v, idx2d, vals2d, cidx2d, zeros_v, ones_f, zf_v,
                  chunk_sh, cnt_sh, sem):
    pltpu.sync_copy(ei_ref.at[pl.ds(_E + tid * _ET, _ET)],
                    dst_v.at[pl.ds(0, _ET)])
    pltpu.sync_copy(ei_ref.at[pl.ds(tid * _ET, _ET)],
                    src_v.at[pl.ds(0, _ET)])

    lane = lax.iota(jnp.int32, 16)
    tail = 16 * (_EIT - 1)
    keep = lane < (_ET - tail)
    dst_v[pl.ds(tail, 16)] = jnp.where(keep, dst_v[pl.ds(tail, 16)], _MARK)
    src_v[pl.ds(tail, 16)] = jnp.where(keep, src_v[pl.ds(tail, 16)], 0)

    def _pre(i, c):
        d = dst_v[pl.ds(i * 16, 16)]
        s = src_v[pl.ds(i * 16, 16)]
        m = jnp.where(s >= _NPC, 1, 0)
        flat_v[pl.ds(i * 16, 16)] = d * _NPC + (s - m * _NPC)
        dump_v[pl.ds(i * 16, 16)] = _CE + (s & (_JUNK - 1))
        vals2d[i // 8, pl.ds((i % 8) * 16, 16)] = 1 + m * 65535
        cidx2d[i // 8, pl.ds((i % 8) * 16, 16)] = d
        return c

    lax.fori_loop(0, _EIT, _pre, 0)

    for j in range(7):
        idx2d[_NDMA - 1, pl.ds(16 + j * 16, 16)] = jnp.full(
            (16,), _CE + j * 16, jnp.int32) + lane
        vals2d[_NDMA - 1, pl.ds(16 + j * 16, 16)] = jnp.zeros(
            (16,), jnp.int32)
        cidx2d[_NDMA - 1, pl.ds(16 + j * 16, 16)] = jnp.full(
            (16,), _MARK, jnp.int32)

    def _chunk(c, carry):
        cb = c * _CE
        plsc.subcore_barrier()
        for z in range(_NZ):
            pltpu.sync_copy(zeros_v,
                            chunk_sh.at[pl.ds(tid * _TZ + z * _ZB, _ZB)])
        plsc.subcore_barrier()

        def _idx(i, cc):
            rel = flat_v[pl.ds(i * 16, 16)] - cb
            ok = (rel >= 0) & (rel < _CE)
            idx2d[i // 8, pl.ds((i % 8) * 16, 16)] = jnp.where(
                ok, rel, dump_v[pl.ds(i * 16, 16)])
            return cc

        lax.fori_loop(0, _EIT, _idx, 0)
        descs = [pltpu.async_copy(vals2d.at[j], chunk_sh.at[idx2d.at[j]],
                                  sem, add=True) for j in range(_NDMA)]
        for dsc in descs:
            dsc.wait()
        plsc.subcore_barrier()
        pltpu.sync_copy(chunk_sh.at[pl.ds(tid * _TZ, _TZ)],
                        a_ref.at[pl.ds(cb + tid * _TZ, _TZ)])
        return carry

    lax.fori_loop(0, _NCH, _chunk, 0)

    plsc.subcore_barrier()
    pltpu.sync_copy(zf_v, cnt_sh.at[pl.ds(tid * _CPT, _CPT)])
    plsc.subcore_barrier()
    descs = [pltpu.async_copy(ones_f, cnt_sh.at[cidx2d.at[j]], sem,
                              add=True) for j in range(_NDMA)]
    for dsc in descs:
        dsc.wait()
    plsc.subcore_barrier()
    pltpu.sync_copy(cnt_sh.at[pl.ds(tid * _CPT, _CPT)],
                    cnt_ref.at[pl.ds(tid * _CPT, _CPT)])


def _sc_build_body(ei_uv_ref, ei_vu_ref, zeros_hbm_ref,
                   a_uv_ref, a_vu_ref, cnt_v_ref,
                   cnt_u_ref, dst_v, src_v, flat_v, dump_v, idx2d, vals2d,
                   cidx2d, zeros_v, ones_f, zf_v, chunk_sh, cnt_sh, sem):
    cid = lax.axis_index("c")
    tid = lax.axis_index("s")

    pltpu.sync_copy(zeros_hbm_ref, zeros_v)
    for j in range(8):
        ones_f[pl.ds(j * 16, 16)] = jnp.ones((16,), jnp.float32)

    def _zf(i, c):
        zf_v[pl.ds(i * 16, 16)] = jnp.zeros((16,), jnp.float32)
        return c

    lax.fori_loop(0, _CPT // 16, _zf, 0)

    args = (tid, dst_v, src_v, flat_v, dump_v, idx2d, vals2d, cidx2d,
            zeros_v, ones_f, zf_v, chunk_sh, cnt_sh, sem)

    @pl.when(cid == 0)
    def _():
        _sc_build_one(ei_uv_ref, a_uv_ref, cnt_v_ref, *args)

    @pl.when(cid == 1)
    def _():
        _sc_build_one(ei_vu_ref, a_vu_ref, cnt_u_ref, *args)


def _sc_build(ei_uv, ei_vu):
    f = pl.kernel(
        _sc_build_body,
        out_type=(
            jax.ShapeDtypeStruct((_NP * _NPC,), jnp.int32),
            jax.ShapeDtypeStruct((_NP * _NPC,), jnp.int32),
            jax.ShapeDtypeStruct((_CNTB,), jnp.float32),
            jax.ShapeDtypeStruct((_CNTB,), jnp.float32),
        ),
        mesh=plsc.VectorSubcoreMesh(core_axis_name="c",
                                    subcore_axis_name="s"),
        scratch_types=[
            pltpu.VMEM((_ETP,), jnp.int32),
            pltpu.VMEM((_ETP,), jnp.int32),
            pltpu.VMEM((_ETP,), jnp.int32),
            pltpu.VMEM((_ETP,), jnp.int32),
            pltpu.VMEM((_NDMA, 128), jnp.int32),
            pltpu.VMEM((_NDMA, 128), jnp.int32),
            pltpu.VMEM((_NDMA, 128), jnp.int32),
            pltpu.VMEM((_ZB,), jnp.int32),
            pltpu.VMEM((128,), jnp.float32),
            pltpu.VMEM((_CPT,), jnp.float32),
            pltpu.VMEM_SHARED((_CBUF,), jnp.int32),
            pltpu.VMEM_SHARED((_CNTB,), jnp.float32),
            pltpu.SemaphoreType.DMA,
        ],
    )
    zeros_hbm = jnp.zeros((_ZB,), jnp.int32)
    return f(ei_uv, ei_vu, zeros_hbm)


def _agg(a_ref, x_ref, cnt_ref):
    a = a_ref[...]
    lo = (a & 0xFFFF).astype(jnp.float32).astype(jnp.bfloat16)
    hi = (a >> 16).astype(jnp.float32).astype(jnp.bfloat16)
    agg = (jnp.dot(lo, x_ref[:_NPC, :], preferred_element_type=jnp.float32)
           + jnp.dot(hi, x_ref[_NPC:, :],
                     preferred_element_type=jnp.float32))
    return (agg / jnp.maximum(cnt_ref[...], 1.0)).astype(jnp.bfloat16)


def _sage_body(a_ref, x_ref, cnt_ref, xd_ref, wl_ref, wr_ref, bl_ref,
               o_ref):
    mean = _agg(a_ref, x_ref, cnt_ref)
    o_ref[...] = (jnp.dot(mean, wl_ref[...],
                          preferred_element_type=jnp.float32)
                  + jnp.dot(xd_ref[...], wr_ref[...],
                            preferred_element_type=jnp.float32)
                  + bl_ref[...]).astype(o_ref.dtype)


def _sage_r_body(a_ref, x_ref, cnt_ref, r_ref, wl_ref, o_ref):
    mean = _agg(a_ref, x_ref, cnt_ref)
    o_ref[...] = (jnp.dot(mean, wl_ref[...],
                          preferred_element_type=jnp.float32)
                  + r_ref[...]).astype(o_ref.dtype)


def _xw_body(xd_ref, wr_ref, bl_ref, o_ref):
    o_ref[...] = (jnp.dot(xd_ref[...], wr_ref[...],
                          preferred_element_type=jnp.float32)
                  + bl_ref[...])


def _xw(x_dst, wr, bias):
    np_, d = x_dst.shape
    return pl.pallas_call(
        _xw_body,
        grid=(np_ // _BM,),
        in_specs=[
            pl.BlockSpec((_BM, d), lambda m: (m, 0)),
            pl.BlockSpec((d, d), lambda m: (0, 0)),
            pl.BlockSpec((1, d), lambda m: (0, 0)),
        ],
        out_specs=pl.BlockSpec((_BM, d), lambda m: (m, 0)),
        out_shape=jax.ShapeDtypeStruct((np_, d), jnp.float32),
        compiler_params=pltpu.CompilerParams(
            dimension_semantics=("arbitrary",)),
    )(x_dst, wr, bias)


def _sage_r(a, x_src, cnt, r, wl, out_dtype):
    np_, d = x_src.shape
    return pl.pallas_call(
        _sage_r_body,
        grid=(np_ // _BM,),
        in_specs=[
            pl.BlockSpec((_BM, _NPC), lambda m: (m, 0)),
            pl.BlockSpec((np_, d), lambda m: (0, 0)),
            pl.BlockSpec((_BM, 1), lambda m: (m, 0)),
            pl.BlockSpec((_BM, d), lambda m: (m, 0)),
            pl.BlockSpec((d, d), lambda m: (0, 0)),
        ],
        out_specs=pl.BlockSpec((_BM, d), lambda m: (m, 0)),
        out_shape=jax.ShapeDtypeStruct((np_, d), out_dtype),
        compiler_params=pltpu.CompilerParams(
            dimension_semantics=("arbitrary",)),
    )(a, x_src, cnt, r, wl)


def _sage(a, x_src, cnt, x_dst, wl, wr, bias, out_dtype):
    np_, d = x_src.shape
    grid = (np_ // _BM,)
    return pl.pallas_call(
        _sage_body,
        grid=grid,
        in_specs=[
            pl.BlockSpec((_BM, _NPC), lambda m: (m, 0)),
            pl.BlockSpec((np_, d), lambda m: (0, 0)),
            pl.BlockSpec((_BM, 1), lambda m: (m, 0)),
            pl.BlockSpec((_BM, d), lambda m: (m, 0)),
            pl.BlockSpec((d, d), lambda m: (0, 0)),
            pl.BlockSpec((d, d), lambda m: (0, 0)),
            pl.BlockSpec((1, d), lambda m: (0, 0)),
        ],
        out_specs=pl.BlockSpec((_BM, d), lambda m: (m, 0)),
        out_shape=jax.ShapeDtypeStruct((np_, d), out_dtype),
        compiler_params=pltpu.CompilerParams(
            dimension_semantics=("arbitrary",)),
    )(a, x_src, cnt, x_dst, wl, wr, bias)


def kernel(x_u, x_v, edge_index_uv, edge_index_vu, Wl, bl, Wr):
    n_u, d = x_u.shape
    n_v = x_v.shape[0]
    np_ = _NP

    xu = jnp.zeros((np_, d), jnp.bfloat16).at[:n_u].set(
        x_u.astype(jnp.bfloat16))
    xv = jnp.zeros((np_, d), jnp.bfloat16).at[:n_v].set(
        x_v.astype(jnp.bfloat16))

    wl16 = Wl.astype(jnp.bfloat16)
    wr16 = Wr.astype(jnp.bfloat16)

    r_v = _xw(xv, wr16[0], bl[0][None, :])
    r_u = _xw(xu, wr16[1], bl[1][None, :])

    a_uv_f, a_vu_f, cnt_v_f, cnt_u_f = _sc_build(
        edge_index_uv.reshape(-1), edge_index_vu.reshape(-1))
    a_uv = a_uv_f.reshape(np_, _NPC)
    a_vu = a_vu_f.reshape(np_, _NPC)
    cnt_v = cnt_v_f[:np_, None]
    cnt_u = cnt_u_f[:np_, None]

    num_layers = Wl.shape[0] // 2
    for i in range(num_layers):
        last = i == num_layers - 1
        odt = jnp.float32 if last else jnp.bfloat16
        if i == 0:
            new_v = _sage_r(a_uv, xu, cnt_v, r_v, wl16[0], odt)
            new_u = _sage_r(a_vu, xv, cnt_u, r_u, wl16[1], odt)
        else:
            new_v = _sage(a_uv, xu, cnt_v, xv, wl16[2 * i], wr16[2 * i],
                          bl[2 * i][None, :], odt)
            new_u = _sage(a_vu, xv, cnt_u, xu, wl16[2 * i + 1],
                          wr16[2 * i + 1], bl[2 * i + 1][None, :], odt)
        xu, xv = new_u, new_v
    return xu[:n_u], xv[:n_v]


# speedup vs baseline: 4.5597x; 1.0375x over previous
import jax
import jax.numpy as jnp
from jax import lax
from jax.experimental import pallas as pl
from jax.experimental.pallas import tpu as pltpu
from jax.experimental.pallas import tpu_sc as plsc

_NP = 5120
_NPC = _NP // 2
_E = 80000
_NT = 16
_ET = _E // _NT
_EIT = 313
_ETP = _EIT * 16
_ROWS = 320
_NCH = _NP // _ROWS
_CE = _ROWS * _NPC
_JUNK = 2048
_CBUF = _CE + _JUNK
_TZ = _CE // _NT
_ZB = 12800
_NZ = _TZ // _ZB
_CNTB = 8192
_CPT = _CNTB // _NT
_NDMA = 40
_MARK = 6000

_BM = 512
_BK = 512


def _sc_build_one(ei_ref, a_ref, cnt_ref, tid, dst_v, src_v, flat_v,
                  dump_v, idx2d, vals2d, cidx2d, zeros_v, ones_f, zf_v,
                  chunk_sh, cnt_sh, sem):
    pltpu.sync_copy(ei_ref.at[pl.ds(_E + tid * _ET, _ET)],
                    dst_v.at[pl.ds(0, _ET)])
    pltpu.sync_copy(ei_ref.at[pl.ds(tid * _ET, _ET)],
                    src_v.at[pl.ds(0, _ET)])

    lane = lax.iota(jnp.int32, 16)
    tail = 16 * (_EIT - 1)
    keep = lane < (_ET - tail)
    dst_v[pl.ds(tail, 16)] = jnp.where(keep, dst_v[pl.ds(tail, 16)], _MARK)
    src_v[pl.ds(tail, 16)] = jnp.where(keep, src_v[pl.ds(tail, 16)], 0)

    def _pre(i, c):
        d = dst_v[pl.ds(i * 16, 16)]
        s = src_v[pl.ds(i * 16, 16)]
        m = jnp.where(s >= _NPC, 1, 0)
        flat_v[pl.ds(i * 16, 16)] = d * _NPC + (s - m * _NPC)
        dump_v[pl.ds(i * 16, 16)] = _CE + (s & (_JUNK - 1))
        vals2d[i // 8, pl.ds((i % 8) * 16, 16)] = 1 + m * 65535
        cidx2d[i // 8, pl.ds((i % 8) * 16, 16)] = d
        return c

    lax.fori_loop(0, _EIT, _pre, 0)

    for j in range(7):
        idx2d[_NDMA - 1, pl.ds(16 + j * 16, 16)] = jnp.full(
            (16,), _CE + j * 16, jnp.int32) + lane
        vals2d[_NDMA - 1, pl.ds(16 + j * 16, 16)] = jnp.zeros(
            (16,), jnp.int32)
        cidx2d[_NDMA - 1, pl.ds(16 + j * 16, 16)] = jnp.full(
            (16,), _MARK, jnp.int32)

    def _chunk(c, carry):
        cb = c * _CE
        plsc.subcore_barrier()
        zds = [pltpu.async_copy(
            zeros_v, chunk_sh.at[pl.ds(tid * _TZ + z * _ZB, _ZB)], sem)
            for z in range(_NZ)]
        for zd in zds:
            zd.wait()
        plsc.subcore_barrier()

        def _idx(i, cc):
            rel = flat_v[pl.ds(i * 16, 16)] - cb
            ok = (rel >= 0) & (rel < _CE)
            idx2d[i // 8, pl.ds((i % 8) * 16, 16)] = jnp.where(
                ok, rel, dump_v[pl.ds(i * 16, 16)])
            return cc

        lax.fori_loop(0, _EIT, _idx, 0)
        descs = [pltpu.async_copy(vals2d.at[j], chunk_sh.at[idx2d.at[j]],
                                  sem, add=True) for j in range(_NDMA)]
        for dsc in descs:
            dsc.wait()
        plsc.subcore_barrier()
        pltpu.sync_copy(chunk_sh.at[pl.ds(tid * _TZ, _TZ)],
                        a_ref.at[pl.ds(cb + tid * _TZ, _TZ)])
        return carry

    lax.fori_loop(0, _NCH, _chunk, 0)

    plsc.subcore_barrier()
    pltpu.sync_copy(zf_v, cnt_sh.at[pl.ds(tid * _CPT, _CPT)])
    plsc.subcore_barrier()
    descs = [pltpu.async_copy(ones_f, cnt_sh.at[cidx2d.at[j]], sem,
                              add=True) for j in range(_NDMA)]
    for dsc in descs:
        dsc.wait()
    plsc.subcore_barrier()
    pltpu.sync_copy(cnt_sh.at[pl.ds(tid * _CPT, _CPT)],
                    cnt_ref.at[pl.ds(tid * _CPT, _CPT)])


def _sc_build_body(ei_uv_ref, ei_vu_ref, zeros_hbm_ref,
                   a_uv_ref, a_vu_ref, cnt_v_ref,
                   cnt_u_ref, dst_v, src_v, flat_v, dump_v, idx2d, vals2d,
                   cidx2d, zeros_v, ones_f, zf_v, chunk_sh, cnt_sh, sem):
    cid = lax.axis_index("c")
    tid = lax.axis_index("s")

    pltpu.sync_copy(zeros_hbm_ref, zeros_v)
    for j in range(8):
        ones_f[pl.ds(j * 16, 16)] = jnp.ones((16,), jnp.float32)

    def _zf(i, c):
        zf_v[pl.ds(i * 16, 16)] = jnp.zeros((16,), jnp.float32)
        return c

    lax.fori_loop(0, _CPT // 16, _zf, 0)

    args = (tid, dst_v, src_v, flat_v, dump_v, idx2d, vals2d, cidx2d,
            zeros_v, ones_f, zf_v, chunk_sh, cnt_sh, sem)

    @pl.when(cid == 0)
    def _():
        _sc_build_one(ei_uv_ref, a_uv_ref, cnt_v_ref, *args)

    @pl.when(cid == 1)
    def _():
        _sc_build_one(ei_vu_ref, a_vu_ref, cnt_u_ref, *args)


def _sc_build(ei_uv, ei_vu):
    f = pl.kernel(
        _sc_build_body,
        out_type=(
            jax.ShapeDtypeStruct((_NP * _NPC,), jnp.int32),
            jax.ShapeDtypeStruct((_NP * _NPC,), jnp.int32),
            jax.ShapeDtypeStruct((_CNTB,), jnp.float32),
            jax.ShapeDtypeStruct((_CNTB,), jnp.float32),
        ),
        mesh=plsc.VectorSubcoreMesh(core_axis_name="c",
                                    subcore_axis_name="s"),
        scratch_types=[
            pltpu.VMEM((_ETP,), jnp.int32),
            pltpu.VMEM((_ETP,), jnp.int32),
            pltpu.VMEM((_ETP,), jnp.int32),
            pltpu.VMEM((_ETP,), jnp.int32),
            pltpu.VMEM((_NDMA, 128), jnp.int32),
            pltpu.VMEM((_NDMA, 128), jnp.int32),
            pltpu.VMEM((_NDMA, 128), jnp.int32),
            pltpu.VMEM((_ZB,), jnp.int32),
            pltpu.VMEM((128,), jnp.float32),
            pltpu.VMEM((_CPT,), jnp.float32),
            pltpu.VMEM_SHARED((_CBUF,), jnp.int32),
            pltpu.VMEM_SHARED((_CNTB,), jnp.float32),
            pltpu.SemaphoreType.DMA,
        ],
    )
    zeros_hbm = jnp.zeros((_ZB,), jnp.int32)
    return f(ei_uv, ei_vu, zeros_hbm)


def _agg(a_ref, x_ref, cnt_ref):
    a = a_ref[...]
    lo = (a & 0xFFFF).astype(jnp.float32).astype(jnp.bfloat16)
    hi = (a >> 16).astype(jnp.float32).astype(jnp.bfloat16)
    agg = (jnp.dot(lo, x_ref[:_NPC, :], preferred_element_type=jnp.float32)
           + jnp.dot(hi, x_ref[_NPC:, :],
                     preferred_element_type=jnp.float32))
    return (agg / jnp.maximum(cnt_ref[...], 1.0)).astype(jnp.bfloat16)


def _sage_body(a_ref, x_ref, cnt_ref, xd_ref, wl_ref, wr_ref, bl_ref,
               o_ref):
    mean = _agg(a_ref, x_ref, cnt_ref)
    o_ref[...] = (jnp.dot(mean, wl_ref[...],
                          preferred_element_type=jnp.float32)
                  + jnp.dot(xd_ref[...], wr_ref[...],
                            preferred_element_type=jnp.float32)
                  + bl_ref[...]).astype(o_ref.dtype)


def _sage_r_body(a_ref, x_ref, cnt_ref, r_ref, wl_ref, o_ref):
    mean = _agg(a_ref, x_ref, cnt_ref)
    o_ref[...] = (jnp.dot(mean, wl_ref[...],
                          preferred_element_type=jnp.float32)
                  + r_ref[...]).astype(o_ref.dtype)


def _xw_body(xd_ref, wr_ref, bl_ref, o_ref):
    o_ref[...] = (jnp.dot(xd_ref[...], wr_ref[...],
                          preferred_element_type=jnp.float32)
                  + bl_ref[...])


def _xw(x_dst, wr, bias):
    np_, d = x_dst.shape
    return pl.pallas_call(
        _xw_body,
        grid=(np_ // _BM,),
        in_specs=[
            pl.BlockSpec((_BM, d), lambda m: (m, 0)),
            pl.BlockSpec((d, d), lambda m: (0, 0)),
            pl.BlockSpec((1, d), lambda m: (0, 0)),
        ],
        out_specs=pl.BlockSpec((_BM, d), lambda m: (m, 0)),
        out_shape=jax.ShapeDtypeStruct((np_, d), jnp.float32),
        compiler_params=pltpu.CompilerParams(
            dimension_semantics=("arbitrary",)),
    )(x_dst, wr, bias)


def _sage_r(a, x_src, cnt, r, wl, out_dtype):
    np_, d = x_src.shape
    return pl.pallas_call(
        _sage_r_body,
        grid=(np_ // _BM,),
        in_specs=[
            pl.BlockSpec((_BM, _NPC), lambda m: (m, 0)),
            pl.BlockSpec((np_, d), lambda m: (0, 0)),
            pl.BlockSpec((_BM, 1), lambda m: (m, 0)),
            pl.BlockSpec((_BM, d), lambda m: (m, 0)),
            pl.BlockSpec((d, d), lambda m: (0, 0)),
        ],
        out_specs=pl.BlockSpec((_BM, d), lambda m: (m, 0)),
        out_shape=jax.ShapeDtypeStruct((np_, d), out_dtype),
        compiler_params=pltpu.CompilerParams(
            dimension_semantics=("arbitrary",)),
    )(a, x_src, cnt, r, wl)


def _sage(a, x_src, cnt, x_dst, wl, wr, bias, out_dtype, out_rows=None):
    np_, d = x_src.shape
    grid = (np_ // _BM,)
    return pl.pallas_call(
        _sage_body,
        grid=grid,
        in_specs=[
            pl.BlockSpec((_BM, _NPC), lambda m: (m, 0)),
            pl.BlockSpec((np_, d), lambda m: (0, 0)),
            pl.BlockSpec((_BM, 1), lambda m: (m, 0)),
            pl.BlockSpec((_BM, d), lambda m: (m, 0)),
            pl.BlockSpec((d, d), lambda m: (0, 0)),
            pl.BlockSpec((d, d), lambda m: (0, 0)),
            pl.BlockSpec((1, d), lambda m: (0, 0)),
        ],
        out_specs=pl.BlockSpec((_BM, d), lambda m: (m, 0)),
        out_shape=jax.ShapeDtypeStruct((out_rows or np_, d), out_dtype),
        compiler_params=pltpu.CompilerParams(
            dimension_semantics=("arbitrary",)),
    )(a, x_src, cnt, x_dst, wl, wr, bias)


def kernel(x_u, x_v, edge_index_uv, edge_index_vu, Wl, bl, Wr):
    n_u, d = x_u.shape
    n_v = x_v.shape[0]
    np_ = _NP

    xu = jnp.zeros((np_, d), jnp.bfloat16).at[:n_u].set(
        x_u.astype(jnp.bfloat16))
    xv = jnp.zeros((np_, d), jnp.bfloat16).at[:n_v].set(
        x_v.astype(jnp.bfloat16))

    wl16 = Wl.astype(jnp.bfloat16)
    wr16 = Wr.astype(jnp.bfloat16)

    r_v = _xw(xv, wr16[0], bl[0][None, :])
    r_u = _xw(xu, wr16[1], bl[1][None, :])

    a_uv_f, a_vu_f, cnt_v_f, cnt_u_f = _sc_build(
        edge_index_uv.reshape(-1), edge_index_vu.reshape(-1))
    a_uv = a_uv_f.reshape(np_, _NPC)
    a_vu = a_vu_f.reshape(np_, _NPC)
    cnt_v = cnt_v_f[:np_, None]
    cnt_u = cnt_u_f[:np_, None]

    num_layers = Wl.shape[0] // 2
    for i in range(num_layers):
        last = i == num_layers - 1
        odt = jnp.float32 if last else jnp.bfloat16
        if i == 0:
            new_v = _sage_r(a_uv, xu, cnt_v, r_v, wl16[0], odt)
            new_u = _sage_r(a_vu, xv, cnt_u, r_u, wl16[1], odt)
        else:
            new_v = _sage(a_uv, xu, cnt_v, xv, wl16[2 * i], wr16[2 * i],
                          bl[2 * i][None, :], odt,
                          out_rows=n_v if last else None)
            new_u = _sage(a_vu, xv, cnt_u, xu, wl16[2 * i + 1],
                          wr16[2 * i + 1], bl[2 * i + 1][None, :], odt,
                          out_rows=n_u if last else None)
        xu, xv = new_u, new_v
    return xu, xv
